# Initial kernel scaffold; baseline (speedup 1.0000x reference)
import jax
import jax.numpy as jnp
from jax.experimental import pallas as pl


def kernel(y_pred, y_true):
    raise NotImplementedError("write your pallas kernel here")


# speedup vs baseline: 28.4012x; 28.4012x over previous
import functools

import jax
import jax.numpy as jnp
from jax import lax
from jax.experimental import pallas as pl
from jax.experimental.pallas import tpu as pltpu
from jax.experimental.pallas import tpu_sc as plsc

N = 16 * 512 * 512
NC, NS = 2, 16
NW = NC * NS
M_PER = N // NW
CHUNK = 8192
NCHUNK = M_PER // CHUNK
VPC = CHUNK // 16
T = 4096
INV_H = 256.0
HIST = 16 * T

def _hist_sc_body(pred_hbm, lab_hbm, hist_out, pacc_out, hist_v, pbuf, lbuf, stage_v):
    c = lax.axis_index("c")
    s = lax.axis_index("s")
    wid = s * NC + c
    base = wid * M_PER

    zeros16 = jnp.zeros((16,), jnp.int32)

    def zbody(i, carry):
        hist_v[pl.ds(i * 16, 16)] = zeros16
        return carry

    lax.fori_loop(0, HIST // 16, zbody, 0)

    lane_off = lax.iota(jnp.int32, 16) * T

    pacc = jnp.zeros((16,), jnp.int32)
    for ci in range(NCHUNK):
        off = base + ci * CHUNK
        pltpu.sync_copy(pred_hbm.at[pl.ds(off, CHUNK)], pbuf)
        pltpu.sync_copy(lab_hbm.at[pl.ds(off, CHUNK)], lbuf)

        def body(i, acc):
            v = pbuf[pl.ds(i * 16, 16)]
            gi = lbuf[pl.ds(i * 16, 16)]
            gf = gi.astype(jnp.float32)
            e = 1.0 - v * (2.0 * gf - 1.0)
            mask = e > 0.0
            b = (e * INV_H).astype(jnp.int32)
            b = jnp.minimum(jnp.maximum(b, 0), T - 1)
            val = 1 + (gi << 16)
            plsc.addupdate_scatter(hist_v, [lane_off + b], val, mask=mask)
            return acc + gi

        pacc = lax.fori_loop(0, VPC, body, pacc)

    stage_v[...] = pacc
    pltpu.sync_copy(hist_v, hist_out.at[wid])
    pltpu.sync_copy(stage_v, pacc_out.at[wid])


@functools.cache
def _hist_sc():
    mesh = plsc.VectorSubcoreMesh(
        core_axis_name="c", subcore_axis_name="s", num_cores=NC, num_subcores=NS
    )
    return pl.kernel(
        _hist_sc_body,
        mesh=mesh,
        out_type=[
            jax.ShapeDtypeStruct((NW, HIST), jnp.int32),
            jax.ShapeDtypeStruct((NW, 16), jnp.int32),
        ],
        scratch_types=[
            pltpu.VMEM((HIST,), jnp.int32),
            pltpu.VMEM((CHUNK,), jnp.float32),
            pltpu.VMEM((CHUNK,), jnp.int32),
            pltpu.VMEM((16,), jnp.int32),
        ],
        compiler_params=pltpu.CompilerParams(needs_layout_passes=False),
    )


def _finish_tc(hist_ref, pacc_ref, out_ref, an_acc, aq_acc):
    g = pl.program_id(0)
    hv = hist_ref[...]
    n3 = jnp.sum((hv & 0xFFFF).astype(jnp.float32), axis=0)
    q3 = jnp.sum((hv >> 16).astype(jnp.float32), axis=0)

    @pl.when(g == 0)
    def _():
        an_acc[...] = n3
        aq_acc[...] = q3

    @pl.when(g > 0)
    def _():
        an_acc[...] += n3
        aq_acc[...] += q3

    @pl.when(g == pl.num_programs(0) - 1)
    def _():
        An = an_acc[...]
        Aq = aq_acc[...]
        r_i = lax.broadcasted_iota(jnp.int32, (128, 128), 0)
        c_i = lax.broadcasted_iota(jnp.int32, (128, 128), 1)
        Mlow = (r_i >= c_i).astype(jnp.float32)
        rr = lax.broadcasted_iota(jnp.int32, (32, 32), 0)
        cc = lax.broadcasted_iota(jnp.int32, (32, 32), 1)
        Tstrict = (cc > rr).astype(jnp.float32)
        ones128 = jnp.ones((128, 128), jnp.float32)
        dot = functools.partial(
            jnp.dot,
            preferred_element_type=jnp.float32,
            precision=lax.Precision.HIGHEST,
        )
        Sn = dot(An, Mlow) + dot(Tstrict, dot(An, ones128))
        Sq = dot(Aq, Mlow) + dot(Tstrict, dot(Aq, ones128))
        P = jnp.sum(pacc_ref[...].astype(jnp.float32))
        J = jnp.where(Sn > 0.5, Sn / (P + Sn - Sq), 0.0)
        rj = lax.broadcasted_iota(jnp.int32, (32, 128), 0)
        cj = lax.broadcasted_iota(jnp.int32, (32, 128), 1)
        j0 = jnp.sum(jnp.where((rj == 0) & (cj == 0), J, 0.0))
        loss = (jnp.sum(J) - 0.5 * j0) * (1.0 / INV_H)
        out_ref[...] = jnp.broadcast_to(loss, (8, 128))


_finish_call = pl.pallas_call(
    _finish_tc,
    grid=(8,),
    in_specs=[
        pl.BlockSpec((64, 32, 128), lambda g: (g, 0, 0)),
        pl.BlockSpec((32, 16), lambda g: (0, 0)),
    ],
    out_specs=pl.BlockSpec((8, 128), lambda g: (0, 0)),
    out_shape=jax.ShapeDtypeStruct((8, 128), jnp.float32),
    scratch_shapes=[
        pltpu.VMEM((32, 128), jnp.float32),
        pltpu.VMEM((32, 128), jnp.float32),
    ],
)


def kernel(y_pred, y_true):
    pred = y_pred.reshape(-1)
    lab = y_true.astype(jnp.int32).reshape(-1)
    hist, pacc = _hist_sc()(pred, lab)
    histr = hist.reshape(NW * 16, 32, 128)
    out = _finish_call(histr, pacc)
    return out[0, 0]


# speedup vs baseline: 37.9953x; 1.3378x over previous
import functools

import jax
import jax.numpy as jnp
from jax import lax
from jax.experimental import pallas as pl
from jax.experimental.pallas import tpu as pltpu
from jax.experimental.pallas import tpu_sc as plsc

NC, NS = 2, 16
NW = NC * NS
NCHUNK = 16
T = 4096
INV_H = 256.0
STRIDE = T + 1
HIST = 16 * STRIDE


def _hist_sc_body(pred_hbm, lab_hbm, n_out, q_out, pacc_out,
                  hist_v, pb0, lb0, pb1, lb1, mn_v, mq_v, stage_v,
                  sp0, sl0, sp1, sl1):
    c = lax.axis_index("c")
    s = lax.axis_index("s")
    wid = s * NC + c
    img = s
    row0 = c * 256

    zeros16 = jnp.zeros((16,), jnp.int32)

    def zbody(i, carry):
        hist_v[pl.ds(i * 16, 16)] = zeros16
        return carry

    lax.fori_loop(0, HIST // 16, zbody, 0)

    lane_base = lax.iota(jnp.int32, 16) * STRIDE

    pbufs = (pb0, pb1)
    lbufs = (lb0, lb1)
    psems = (sp0, sp1)
    lsems = (sl0, sl1)

    def start(ci):
        r = row0 + ci * 16
        cp = pltpu.async_copy(
            pred_hbm.at[img, pl.ds(r, 16)], pbufs[ci % 2], psems[ci % 2])
        cl = pltpu.async_copy(
            lab_hbm.at[img, pl.ds(r, 16)], lbufs[ci % 2], lsems[ci % 2])
        return cp, cl

    pend = start(0)

    pacc = jnp.zeros((16,), jnp.int32)
    for ci in range(NCHUNK):
        pend[0].wait()
        pend[1].wait()
        if ci + 1 < NCHUNK:
            nxt = start(ci + 1)
        pb = pbufs[ci % 2]
        lb = lbufs[ci % 2]

        def body(i, acc, pb=pb, lb=lb):
            off = i * 16
            gs = []
            for r in range(16):
                v = pb[r, pl.ds(off, 16)]
                gi = lb[r, pl.ds(off, 16)]
                gf = gi.astype(jnp.float32)
                e = 1.0 - v * (2.0 * gf - 1.0)
                mask = e > 0.0
                b = (e * INV_H).astype(jnp.int32)
                b = jnp.minimum(b, T - 1)
                val = 1 + (gi << 16)
                plsc.addupdate_scatter(hist_v, [lane_base + b], val, mask=mask)
                gs.append(gi)
            while len(gs) > 1:
                gs = [a + b2 for a, b2 in zip(gs[::2], gs[1::2])]
            return acc + gs[0]

        pacc = lax.fori_loop(0, 32, body, pacc)
        if ci + 1 < NCHUNK:
            pend = nxt

    for r in range(32):
        def mbody(j, carry, r=r):
            base = r * 128 + j * 16
            accn = jnp.zeros((16,), jnp.int32)
            accq = jnp.zeros((16,), jnp.int32)
            for l in range(16):
                v = hist_v[pl.ds(l * STRIDE + base, 16)]
                accn += v & 0xFFFF
                accq += v >> 16
            mn_v[r, pl.ds(j * 16, 16)] = accn
            mq_v[r, pl.ds(j * 16, 16)] = accq
            return carry

        lax.fori_loop(0, 8, mbody, 0)

    stage_v[...] = pacc
    pltpu.sync_copy(mn_v, n_out.at[wid])
    pltpu.sync_copy(mq_v, q_out.at[wid])
    pltpu.sync_copy(stage_v, pacc_out.at[wid])


@functools.cache
def _hist_sc():
    mesh = plsc.VectorSubcoreMesh(
        core_axis_name="c", subcore_axis_name="s", num_cores=NC, num_subcores=NS
    )
    return pl.kernel(
        _hist_sc_body,
        mesh=mesh,
        out_type=[
            jax.ShapeDtypeStruct((NW, 32, 128), jnp.int32),
            jax.ShapeDtypeStruct((NW, 32, 128), jnp.int32),
            jax.ShapeDtypeStruct((NW, 16), jnp.int32),
        ],
        scratch_types=[
            pltpu.VMEM((HIST,), jnp.int32),
            pltpu.VMEM((16, 512), jnp.float32),
            pltpu.VMEM((16, 512), jnp.int32),
            pltpu.VMEM((16, 512), jnp.float32),
            pltpu.VMEM((16, 512), jnp.int32),
            pltpu.VMEM((32, 128), jnp.int32),
            pltpu.VMEM((32, 128), jnp.int32),
            pltpu.VMEM((16,), jnp.int32),
            pltpu.SemaphoreType.DMA,
            pltpu.SemaphoreType.DMA,
            pltpu.SemaphoreType.DMA,
            pltpu.SemaphoreType.DMA,
        ],
        compiler_params=pltpu.CompilerParams(needs_layout_passes=False),
    )


def _finish_tc(n_ref, q_ref, pacc_ref, out_ref):
    An = jnp.sum(n_ref[...].astype(jnp.float32), axis=0)
    Aq = jnp.sum(q_ref[...].astype(jnp.float32), axis=0)
    r_i = lax.broadcasted_iota(jnp.int32, (128, 128), 0)
    c_i = lax.broadcasted_iota(jnp.int32, (128, 128), 1)
    Mlow = (r_i >= c_i).astype(jnp.float32)
    rr = lax.broadcasted_iota(jnp.int32, (32, 32), 0)
    cc = lax.broadcasted_iota(jnp.int32, (32, 32), 1)
    Tstrict = (cc > rr).astype(jnp.float32)
    ones128 = jnp.ones((128, 128), jnp.float32)
    dot = functools.partial(
        jnp.dot,
        preferred_element_type=jnp.float32,
        precision=lax.Precision.HIGHEST,
    )
    Sn = dot(An, Mlow) + dot(Tstrict, dot(An, ones128))
    Sq = dot(Aq, Mlow) + dot(Tstrict, dot(Aq, ones128))
    P = jnp.sum(pacc_ref[...].astype(jnp.float32))
    J = jnp.where(Sn > 0.5, Sn / (P + Sn - Sq), 0.0)
    rj = lax.broadcasted_iota(jnp.int32, (32, 128), 0)
    cj = lax.broadcasted_iota(jnp.int32, (32, 128), 1)
    j0 = jnp.sum(jnp.where((rj == 0) & (cj == 0), J, 0.0))
    loss = (jnp.sum(J) - 0.5 * j0) * (1.0 / INV_H)
    out_ref[...] = jnp.broadcast_to(loss, (8, 128))


_finish_call = pl.pallas_call(
    _finish_tc,
    out_shape=jax.ShapeDtypeStruct((8, 128), jnp.float32),
)


def kernel(y_pred, y_true):
    lab = y_true.astype(jnp.int32)
    nh, qh, pacc = _hist_sc()(y_pred, lab)
    out = _finish_call(nh, qh, pacc)
    return out[0, 0]
